# baseline (speedup 1.0000x reference)
import jax
import jax.numpy as jnp
from jax import lax
from jax.experimental import pallas as pl
from jax.experimental.pallas import tpu as pltpu
from jax.experimental.pallas import tpu_sc as plsc

BATCH = 16384
SEQ = 50
SEQP = 64
EMBED = 64
NUM_CLASSES = 100

NC, NS = 2, 16
NW = NC * NS
BPW = BATCH // NW
BPG = 2
IDX_PER_GATHER = BPG * SEQP
NGATHER = BPW // BPG
NBUF = 4
LANES = 16
VPR = EMBED // LANES


def _pool_body(x_hbm, tbl_hbm, out_hbm, idx_v, pooled_v, bufs, sems):
    wid = lax.axis_index("c") * NS + lax.axis_index("s")
    ipw = BPW * SEQP

    pltpu.sync_copy(x_hbm.at[pl.ds(wid * ipw, ipw)], idx_v)

    def fire(g, b):
        return pltpu.async_copy(
            tbl_hbm.at[idx_v.at[pl.ds(g * IDX_PER_GATHER, IDX_PER_GATHER)]],
            bufs[b], sems[b],
        )

    def wait(g, b):
        pltpu.make_async_copy(
            tbl_hbm.at[idx_v.at[pl.ds(g * IDX_PER_GATHER, IDX_PER_GATHER)]],
            bufs[b], sems[b],
        ).wait()

    for b in range(NBUF):
        fire(b, b)

    @pl.loop(0, NGATHER, step=NBUF)
    def _(g0):
        for b in range(NBUF):
            g = g0 + b
            wait(g, b)
            buf = bufs[b]
            for be in range(BPG):
                accs = [buf[be * SEQP, pl.ds(c * LANES, LANES)] for c in range(VPR)]
                for s in range(1, SEQ):
                    for c in range(VPR):
                        accs[c] += buf[be * SEQP + s, pl.ds(c * LANES, LANES)]
                row = BPG * g + be
                for c in range(VPR):
                    pooled_v[row, pl.ds(c * LANES, LANES)] = accs[c] * (1.0 / SEQ)

            @pl.when(g + NBUF < NGATHER)
            def _():
                fire(g + NBUF, b)

    pltpu.sync_copy(pooled_v, out_hbm.at[pl.ds(wid * BPW, BPW)])


def _sc_pool(x_flat, emb_table):
    scratch = [
        pltpu.VMEM((BPW * SEQP,), jnp.int32),
        pltpu.VMEM((BPW, EMBED), jnp.float32),
        [pltpu.VMEM((IDX_PER_GATHER, EMBED), jnp.float32) for _ in range(NBUF)],
        [pltpu.SemaphoreType.DMA for _ in range(NBUF)],
    ]
    k = pl.kernel(
        _pool_body,
        out_type=jax.ShapeDtypeStruct((BATCH, EMBED), jnp.float32),
        mesh=plsc.VectorSubcoreMesh(
            core_axis_name="c", subcore_axis_name="s",
            num_cores=NC, num_subcores=NS,
        ),
        scratch_types=scratch,
        compiler_params=pltpu.CompilerParams(
            use_tc_tiling_on_sc=False, needs_layout_passes=False
        ),
    )
    return k(x_flat, emb_table)


def _mm_body(p_ref, w_ref, b_ref, o_ref):
    o_ref[...] = (
        jnp.dot(p_ref[...], w_ref[...], preferred_element_type=jnp.float32)
        + b_ref[...]
    )


def _fc(pooled, fc_w, fc_b2):
    blk = 1024
    grid = (BATCH // blk,)
    return pl.pallas_call(
        _mm_body,
        grid=grid,
        in_specs=[
            pl.BlockSpec((blk, EMBED), lambda i: (i, 0)),
            pl.BlockSpec((EMBED, NUM_CLASSES), lambda i: (0, 0)),
            pl.BlockSpec((1, NUM_CLASSES), lambda i: (0, 0)),
        ],
        out_specs=pl.BlockSpec((blk, NUM_CLASSES), lambda i: (i, 0)),
        out_shape=jax.ShapeDtypeStruct((BATCH, NUM_CLASSES), jnp.float32),
    )(pooled, fc_w, fc_b2)


def kernel(x, emb_table, fc_w, fc_b):
    xp = jnp.pad(x, ((0, 0), (0, SEQP - SEQ))).reshape(-1)
    pooled = _sc_pool(xp, emb_table)
    return _fc(pooled, fc_w, fc_b.reshape(1, NUM_CLASSES))


# speedup vs baseline: 5.9043x; 5.9043x over previous
import jax
import jax.numpy as jnp
from jax import lax
from jax.experimental import pallas as pl
from jax.experimental.pallas import tpu as pltpu
from jax.experimental.pallas import tpu_sc as plsc

BATCH = 16384
SEQ = 50
EMBED = 64
NUM_CLASSES = 100

NC, NS = 2, 16
NW = NC * NS
BPW = BATCH // NW
BPG = 4
IDX_PER_GATHER = BPG * SEQ
NGATHER = BPW // BPG
NBUF = 4
LANES = 16
VPR = EMBED // LANES


REPACK_BLK = 512


def _repack_body(x_ref, o_ref):
    for b in range(REPACK_BLK):
        o_ref[pl.ds(b * SEQ, SEQ)] = x_ref[b, :]


def _repack(x):
    grid = (BATCH // REPACK_BLK,)
    return pl.pallas_call(
        _repack_body,
        grid=grid,
        in_specs=[pl.BlockSpec((REPACK_BLK, SEQ), lambda i: (i, 0))],
        out_specs=pl.BlockSpec((REPACK_BLK * SEQ,), lambda i: (i,)),
        out_shape=jax.ShapeDtypeStruct((BATCH * SEQ,), jnp.int32),
    )(x)


def _pool_body(x_hbm, tbl_hbm, out_hbm, idx_v, pooled_v, bufs, sems):
    wid = lax.axis_index("c") * NS + lax.axis_index("s")

    ipw = BPW * SEQ
    pltpu.sync_copy(x_hbm.at[pl.ds(wid * ipw, ipw)], idx_v)

    def fire(g, b):
        return pltpu.async_copy(
            tbl_hbm.at[idx_v.at[pl.ds(g * IDX_PER_GATHER, IDX_PER_GATHER)]],
            bufs[b], sems[b],
        )

    def wait(g, b):
        pltpu.make_async_copy(
            tbl_hbm.at[idx_v.at[pl.ds(g * IDX_PER_GATHER, IDX_PER_GATHER)]],
            bufs[b], sems[b],
        ).wait()

    for b in range(NBUF):
        fire(b, b)

    @pl.loop(0, NGATHER, step=NBUF)
    def _(g0):
        for b in range(NBUF):
            g = g0 + b
            wait(g, b)
            buf = bufs[b]
            for be in range(BPG):
                accs = [buf[be * SEQ, pl.ds(c * LANES, LANES)] for c in range(VPR)]
                for s in range(1, SEQ):
                    for c in range(VPR):
                        accs[c] += buf[be * SEQ + s, pl.ds(c * LANES, LANES)]
                row = BPG * g + be
                for c in range(VPR):
                    pooled_v[row, pl.ds(c * LANES, LANES)] = accs[c] * (1.0 / SEQ)

            @pl.when(g + NBUF < NGATHER)
            def _():
                fire(g + NBUF, b)

    pltpu.sync_copy(pooled_v, out_hbm.at[pl.ds(wid * BPW, BPW)])


def _sc_pool(xr, emb_table):
    scratch = [
        pltpu.VMEM((BPW * SEQ,), jnp.int32),
        pltpu.VMEM((BPW, EMBED), jnp.float32),
        [pltpu.VMEM((IDX_PER_GATHER, EMBED), jnp.float32) for _ in range(NBUF)],
        [pltpu.SemaphoreType.DMA for _ in range(NBUF)],
    ]
    k = pl.kernel(
        _pool_body,
        out_type=jax.ShapeDtypeStruct((BATCH, EMBED), jnp.float32),
        mesh=plsc.VectorSubcoreMesh(
            core_axis_name="c", subcore_axis_name="s",
            num_cores=NC, num_subcores=NS,
        ),
        scratch_types=scratch,
        compiler_params=pltpu.CompilerParams(
            use_tc_tiling_on_sc=False, needs_layout_passes=False
        ),
    )
    return k(xr, emb_table)


def _mm_body(p_ref, w_ref, b_ref, o_ref):
    o_ref[...] = (
        jnp.dot(p_ref[...], w_ref[...], preferred_element_type=jnp.float32)
        + b_ref[...]
    )


def _fc(pooled, fc_w, fc_b2):
    blk = 1024
    grid = (BATCH // blk,)
    return pl.pallas_call(
        _mm_body,
        grid=grid,
        in_specs=[
            pl.BlockSpec((blk, EMBED), lambda i: (i, 0)),
            pl.BlockSpec((EMBED, NUM_CLASSES), lambda i: (0, 0)),
            pl.BlockSpec((1, NUM_CLASSES), lambda i: (0, 0)),
        ],
        out_specs=pl.BlockSpec((blk, NUM_CLASSES), lambda i: (i, 0)),
        out_shape=jax.ShapeDtypeStruct((BATCH, NUM_CLASSES), jnp.float32),
    )(pooled, fc_w, fc_b2)


def kernel(x, emb_table, fc_w, fc_b):
    xr = _repack(x)
    pooled = _sc_pool(xr, emb_table)
    return _fc(pooled, fc_w, fc_b.reshape(1, NUM_CLASSES))


# speedup vs baseline: 7.0058x; 1.1866x over previous
import jax
import jax.numpy as jnp
from jax import lax
from jax.experimental import pallas as pl
from jax.experimental.pallas import tpu as pltpu
from jax.experimental.pallas import tpu_sc as plsc

BATCH = 16384
SEQ = 50
EMBED = 64
NUM_CLASSES = 100

NC, NS = 2, 16
NW = NC * NS
BPW = BATCH // NW
BPG = 4
IDX_PER_GATHER = BPG * SEQ
NGATHER = BPW // BPG
NBUF = 5
LANES = 16
VPR = EMBED // LANES


REPACK_BLK = 512


def _repack_body(x_ref, o_ref):
    for b in range(REPACK_BLK):
        o_ref[pl.ds(b * SEQ, SEQ)] = x_ref[b, :]


def _repack(x):
    grid = (BATCH // REPACK_BLK,)
    return pl.pallas_call(
        _repack_body,
        grid=grid,
        in_specs=[pl.BlockSpec((REPACK_BLK, SEQ), lambda i: (i, 0))],
        out_specs=pl.BlockSpec((REPACK_BLK * SEQ,), lambda i: (i,)),
        out_shape=jax.ShapeDtypeStruct((BATCH * SEQ,), jnp.int32),
    )(x)


def _pool_body(x_hbm, tbl_hbm, out_hbm, idx_v, pooled_v, bufs, sems):
    wid = lax.axis_index("c") * NS + lax.axis_index("s")

    ipw = BPW * SEQ
    pltpu.sync_copy(x_hbm.at[pl.ds(wid * ipw, ipw)], idx_v)

    def fire(g, b):
        return pltpu.async_copy(
            tbl_hbm.at[idx_v.at[pl.ds(g * IDX_PER_GATHER, IDX_PER_GATHER)]],
            bufs[b], sems[b],
        )

    def wait(g, b):
        pltpu.make_async_copy(
            tbl_hbm.at[idx_v.at[pl.ds(g * IDX_PER_GATHER, IDX_PER_GATHER)]],
            bufs[b], sems[b],
        ).wait()

    def consume(g, b):
        buf = bufs[b]
        for be in range(BPG):
            init = tuple(buf[be * SEQ, pl.ds(c * LANES, LANES)] for c in range(VPR))

            @pl.loop(1, SEQ, init_carry=init, unroll=7)
            def accs(s, carry):
                return tuple(
                    carry[c] + buf[be * SEQ + s, pl.ds(c * LANES, LANES)]
                    for c in range(VPR)
                )

            row = BPG * g + be
            for c in range(VPR):
                pooled_v[row, pl.ds(c * LANES, LANES)] = accs[c] * (1.0 / SEQ)

    for b in range(NBUF):
        fire(b, b)

    main_upper = (NGATHER // NBUF) * NBUF

    @pl.loop(0, main_upper, step=NBUF)
    def _(g0):
        for b in range(NBUF):
            g = g0 + b
            wait(g, b)
            consume(g, b)

            @pl.when(g + NBUF < NGATHER)
            def _():
                fire(g + NBUF, b)

    for t in range(main_upper, NGATHER):
        wait(t, t % NBUF)
        consume(t, t % NBUF)

    pltpu.sync_copy(pooled_v, out_hbm.at[pl.ds(wid * BPW, BPW)])


def _sc_pool(xr, emb_table):
    scratch = [
        pltpu.VMEM((BPW * SEQ,), jnp.int32),
        pltpu.VMEM((BPW, EMBED), jnp.float32),
        [pltpu.VMEM((IDX_PER_GATHER, EMBED), jnp.float32) for _ in range(NBUF)],
        [pltpu.SemaphoreType.DMA for _ in range(NBUF)],
    ]
    k = pl.kernel(
        _pool_body,
        out_type=jax.ShapeDtypeStruct((BATCH, EMBED), jnp.float32),
        mesh=plsc.VectorSubcoreMesh(
            core_axis_name="c", subcore_axis_name="s",
            num_cores=NC, num_subcores=NS,
        ),
        scratch_types=scratch,
        compiler_params=pltpu.CompilerParams(
            use_tc_tiling_on_sc=False, needs_layout_passes=False
        ),
    )
    return k(xr, emb_table)


def _mm_body(p_ref, w_ref, b_ref, o_ref):
    o_ref[...] = (
        jnp.dot(p_ref[...], w_ref[...], preferred_element_type=jnp.float32)
        + b_ref[...]
    )


def _fc(pooled, fc_w, fc_b2):
    blk = 1024
    grid = (BATCH // blk,)
    return pl.pallas_call(
        _mm_body,
        grid=grid,
        in_specs=[
            pl.BlockSpec((blk, EMBED), lambda i: (i, 0)),
            pl.BlockSpec((EMBED, NUM_CLASSES), lambda i: (0, 0)),
            pl.BlockSpec((1, NUM_CLASSES), lambda i: (0, 0)),
        ],
        out_specs=pl.BlockSpec((blk, NUM_CLASSES), lambda i: (i, 0)),
        out_shape=jax.ShapeDtypeStruct((BATCH, NUM_CLASSES), jnp.float32),
    )(pooled, fc_w, fc_b2)


def kernel(x, emb_table, fc_w, fc_b):
    xr = _repack(x)
    pooled = _sc_pool(xr, emb_table)
    return _fc(pooled, fc_w, fc_b.reshape(1, NUM_CLASSES))
